# baseline (speedup 1.0000x reference)
import functools

import jax
import jax.numpy as jnp
from jax import lax
from jax.experimental import pallas as pl
from jax.experimental.pallas import tpu as pltpu
from jax.experimental.pallas import tpu_sc as plsc

_T = 2048
_H = 1024
_E = 8
_I = 2048
_BT = 128
_P = 5120
_NTB = _P // _BT
_TPW = _T // 16
_NP = _T * 2
_RPW = _P // 16
_RPW32 = _P // 32
_CH = 40
_CTOK = _T // 32

_mesh = plsc.VectorSubcoreMesh(core_axis_name="c", subcore_axis_name="s")


def _iota16():
    return lax.iota(jnp.int32, 16)


@functools.partial(
    pl.kernel, mesh=_mesh,
    compiler_params=pltpu.CompilerParams(needs_layout_passes=False),
    out_type=[
        jax.ShapeDtypeStruct((_P,), jnp.int32),
        jax.ShapeDtypeStruct((_T,), jnp.int32),
        jax.ShapeDtypeStruct((_T,), jnp.int32),
        jax.ShapeDtypeStruct((_T,), jnp.float32),
        jax.ShapeDtypeStruct((_T,), jnp.float32),
        jax.ShapeDtypeStruct((64,), jnp.int32),
    ],
    scratch_types=[
        pltpu.VMEM((_TPW * _E,), jnp.float32),
        pltpu.VMEM((_TPW,), jnp.int32),
        pltpu.VMEM((_TPW,), jnp.int32),
        pltpu.VMEM((_TPW,), jnp.float32),
        pltpu.VMEM((_TPW,), jnp.float32),
        pltpu.VMEM((_TPW,), jnp.int32),
        pltpu.VMEM((_TPW,), jnp.int32),
        pltpu.VMEM((16,), jnp.float32),
        pltpu.VMEM((16,), jnp.int32),
        pltpu.VMEM((32,), jnp.int32),
        pltpu.VMEM((32,), jnp.int32),
        pltpu.VMEM((256,), jnp.int32),
        pltpu.VMEM((2 * _TPW,), jnp.int32),
        pltpu.VMEM((2 * _TPW,), jnp.int32),
        pltpu.VMEM((_NP,), jnp.int32),
        pltpu.VMEM((_NP,), jnp.int32),
        pltpu.VMEM((_RPW,), jnp.int32),
        pltpu.VMEM((64,), jnp.int32),
        pltpu.VMEM_SHARED((_NP,), jnp.int32),
        pltpu.VMEM_SHARED((_NP,), jnp.int32),
        pltpu.VMEM_SHARED((256,), jnp.int32),
    ],
)
def _route_sort(lg_hbm, flag_hbm, tok_pad_hbm, pos1_hbm, pos2_hbm, w1_hbm,
                w2_hbm, te_hbm, lg_v, e1_v, e2_v, p1_v, p2_v, pos1_v, pos2_v,
                flag_v, cnt_v, base_v, ends_v, allcnt_v, mypos_v, mytok_v,
                allpos_v, alltok_v, tokslice_v, te_v, pos_sh, tok_sh, cnt_sh):
    core = lax.axis_index("c")
    sub = lax.axis_index("s")
    iota = _iota16()
    wbase_t = sub * _TPW

    def body():
        pltpu.sync_copy(lg_hbm.at[pl.ds(wbase_t * _E, _TPW * _E)], lg_v)
        pltpu.sync_copy(flag_hbm, flag_v)
        sel = flag_v[...] > 0.0
        for g in range(_TPW // 16):
            row = (jnp.full((16,), g * 16, jnp.int32) + iota) * _E
            lgs = [plsc.load_gather(
                lg_v, [row + jnp.full((16,), e, jnp.int32)])
                for e in range(_E)]
            m1 = lgs[0]
            i1 = jnp.zeros((16,), jnp.int32)
            for e in range(1, _E):
                take = lgs[e] > m1
                m1 = jnp.where(take, lgs[e], m1)
                i1 = jnp.where(take, e, i1)
            m2 = jnp.full((16,), -jnp.inf, jnp.float32)
            i2 = jnp.zeros((16,), jnp.int32)
            for e in range(_E):
                take = (i1 != e) & (lgs[e] > m2)
                m2 = jnp.where(take, lgs[e], m2)
                i2 = jnp.where(take, e, i2)
            s = jnp.zeros((16,), jnp.float32)
            for e in range(_E):
                s = s + jnp.exp(lgs[e] - m1)
            er = jnp.exp(m2 - m1)
            p1r = 1.0 / (1.0 + er)
            p1 = jnp.where(sel, p1r, 1.0 / s)
            p2 = jnp.where(sel, 1.0 - p1r, er / s)
            e1_v[pl.ds(g * 16, 16)] = i1
            e2_v[pl.ds(g * 16, 16)] = i2
            p1_v[pl.ds(g * 16, 16)] = p1
            p2_v[pl.ds(g * 16, 16)] = p2

        cnt = jnp.zeros((16,), jnp.int32)
        for g in range(_TPW // 16):
            for src in (e1_v, e2_v):
                v = src[pl.ds(g * 16, 16)]
                for e in range(_E):
                    c = plsc.all_reduce_population_count(v == e)
                    cnt = cnt + jnp.where(iota == e, c, 0)
        cnt_v[...] = cnt
        pltpu.sync_copy(cnt_v, cnt_sh.at[pl.ds(sub * 16, 16)])
        plsc.subcore_barrier()

        pltpu.sync_copy(cnt_sh, allcnt_v)
        sub_sp = jnp.full((16,), sub, jnp.int32)
        total = jnp.zeros((16,), jnp.int32)
        prefix = jnp.zeros((16,), jnp.int32)
        for w in range(16):
            v = allcnt_v[pl.ds(w * 16, 16)]
            total = total + v
            prefix = prefix + jnp.where(
                jnp.full((16,), w, jnp.int32) < sub_sp, v, 0)
        padded = ((total + (_BT - 1)) >> 7) << 7
        incl = plsc.cumsum(padded)
        off = incl - padded
        basev = off + prefix
        base_v[pl.ds(0, 16)] = basev
        base_v[pl.ds(16, 16)] = basev
        base = [plsc.load_gather(base_v, [jnp.full((16,), 16 + e, jnp.int32)])
                for e in range(_E)]

        for g in range(_TPW // 16):
            row_tok = jnp.full((16,), wbase_t + g * 16, jnp.int32) + iota
            for k, (ev, posv) in enumerate(((e1_v, pos1_v), (e2_v, pos2_v))):
                v = ev[pl.ds(g * 16, 16)]
                pos = jnp.zeros((16,), jnp.int32)
                for e in range(_E):
                    m = v == e
                    mi = jnp.where(m, 1, 0)
                    rk = plsc.cumsum(mi) - mi
                    pos = jnp.where(m, base[e] + rk, pos)
                    base[e] = base[e] + plsc.all_reduce_population_count(m)
                posv[pl.ds(g * 16, 16)] = pos
                mypos_v[pl.ds(k * _TPW + g * 16, 16)] = pos
                mytok_v[pl.ds(k * _TPW + g * 16, 16)] = row_tok
        pltpu.sync_copy(mypos_v, pos_sh.at[pl.ds(sub * 2 * _TPW, 2 * _TPW)])
        pltpu.sync_copy(mytok_v, tok_sh.at[pl.ds(sub * 2 * _TPW, 2 * _TPW)])
        plsc.subcore_barrier()

        pltpu.sync_copy(pos_sh, allpos_v)
        pltpu.sync_copy(tok_sh, alltok_v)
        for j in range(_RPW // 16):
            tokslice_v[pl.ds(j * 16, 16)] = jnp.zeros((16,), jnp.int32)
        lo = sub * _RPW
        losp = jnp.full((16,), lo, jnp.int32)
        hisp = losp + _RPW

        def scan_pairs(j, carry):
            pv = allpos_v[pl.ds(j * 16, 16)]
            tv = alltok_v[pl.ds(j * 16, 16)]
            m = (pv >= losp) & (pv < hisp)
            plsc.store_scatter(tokslice_v, [pv - losp], tv, mask=m)
            return carry

        lax.fori_loop(0, _NP // 16, scan_pairs, 0)

        @pl.when(core == 0)
        def _():
            pltpu.sync_copy(tokslice_v, tok_pad_hbm.at[pl.ds(lo, _RPW)])
            pltpu.sync_copy(pos1_v, pos1_hbm.at[pl.ds(wbase_t, _TPW)])
            pltpu.sync_copy(pos2_v, pos2_hbm.at[pl.ds(wbase_t, _TPW)])
            pltpu.sync_copy(p1_v, w1_hbm.at[pl.ds(wbase_t, _TPW)])
            pltpu.sync_copy(p2_v, w2_hbm.at[pl.ds(wbase_t, _TPW)])

        @pl.when((core == 0) & (sub == 0))
        def _():
            ends_v[pl.ds(0, 16)] = incl
            ends_v[pl.ds(16, 16)] = incl
            for v3 in range(4):
                jt = (jnp.full((16,), v3 * 16, jnp.int32) + iota) * _BT
                acc = jnp.zeros((16,), jnp.int32)
                for e in range(_E):
                    end_e = plsc.load_gather(
                        ends_v, [jnp.full((16,), 16 + e, jnp.int32)])
                    acc = acc + jnp.where(jt >= end_e, 1, 0)
                te_v[pl.ds(v3 * 16, 16)] = jnp.minimum(acc, _E - 1)
            pltpu.sync_copy(te_v, te_hbm)

    body()


@functools.partial(
    pl.kernel, mesh=_mesh,
    compiler_params=pltpu.CompilerParams(needs_layout_passes=False),
    out_type=jax.ShapeDtypeStruct((_P, _H), jnp.float32),
    scratch_types=[
        pltpu.VMEM((_CH,), jnp.int32),
        pltpu.VMEM((_CH, _H), jnp.float32),
        pltpu.SemaphoreType.DMA,
    ],
)
def _gather_rows(hid_hbm, tok_hbm, xs_hbm, idx_v, rows_v, sem):
    core = lax.axis_index("c")
    sub = lax.axis_index("s")
    wid = sub * 2 + core
    base = wid * _RPW32
    for ch in range(_RPW32 // _CH):
        pltpu.sync_copy(tok_hbm.at[pl.ds(base + ch * _CH, _CH)], idx_v)
        pltpu.async_copy(hid_hbm.at[idx_v], rows_v, sem).wait()
        pltpu.sync_copy(rows_v, xs_hbm.at[pl.ds(base + ch * _CH, _CH)])


def _mlp_body(te_ref, x_ref, w13_ref, w2_ref, out_ref):
    x = x_ref[...]
    w1 = w13_ref[0, pl.ds(0, _I), :]
    w3 = w13_ref[0, pl.ds(_I, _I), :]
    w2 = w2_ref[0]
    dn = (((1,), (1,)), ((), ()))
    gate = lax.dot_general(x, w1, dn, preferred_element_type=jnp.float32)
    up = lax.dot_general(x, w3, dn, preferred_element_type=jnp.float32)
    act = gate * lax.logistic(gate) * up
    out_ref[...] = lax.dot_general(act, w2, dn,
                                   preferred_element_type=jnp.float32)


def _grouped_mlp(te, xs, W13, W2):
    grid_spec = pltpu.PrefetchScalarGridSpec(
        num_scalar_prefetch=1,
        grid=(_NTB,),
        in_specs=[
            pl.BlockSpec((_BT, _H), lambda t, te_ref: (t, 0)),
            pl.BlockSpec((1, 2 * _I, _H), lambda t, te_ref: (te_ref[t], 0, 0)),
            pl.BlockSpec((1, _H, _I), lambda t, te_ref: (te_ref[t], 0, 0)),
        ],
        out_specs=pl.BlockSpec((_BT, _H), lambda t, te_ref: (t, 0)),
    )
    return pl.pallas_call(
        _mlp_body,
        grid_spec=grid_spec,
        out_shape=jax.ShapeDtypeStruct((_P, _H), jnp.float32),
    )(te, xs, W13, W2)


@functools.partial(
    pl.kernel, mesh=_mesh,
    compiler_params=pltpu.CompilerParams(needs_layout_passes=False),
    out_type=jax.ShapeDtypeStruct((_T, _H), jnp.float32),
    scratch_types=[
        pltpu.VMEM((16,), jnp.int32),
        pltpu.VMEM((16,), jnp.int32),
        pltpu.VMEM((2 * _CTOK,), jnp.float32),
        pltpu.VMEM((2 * _CTOK,), jnp.float32),
        pltpu.VMEM((16, _H), jnp.float32),
        pltpu.VMEM((16, _H), jnp.float32),
        pltpu.VMEM((16, _H), jnp.float32),
        pltpu.SemaphoreType.DMA,
        pltpu.SemaphoreType.DMA,
    ],
)
def _combine(outs_hbm, pos1_hbm, pos2_hbm, w1_hbm, w2_hbm, fin_hbm,
             p1c, p2c, w1s_v, w2s_v, g1, g2, res, sem1, sem2):
    core = lax.axis_index("c")
    sub = lax.axis_index("s")
    wid = sub * 2 + core
    tbase = wid * _CTOK
    pltpu.sync_copy(w1_hbm.at[pl.ds(tbase, _CTOK)], w1s_v.at[pl.ds(0, _CTOK)])
    pltpu.sync_copy(w1_hbm.at[pl.ds(tbase, _CTOK)],
                    w1s_v.at[pl.ds(_CTOK, _CTOK)])
    pltpu.sync_copy(w2_hbm.at[pl.ds(tbase, _CTOK)], w2s_v.at[pl.ds(0, _CTOK)])
    pltpu.sync_copy(w2_hbm.at[pl.ds(tbase, _CTOK)],
                    w2s_v.at[pl.ds(_CTOK, _CTOK)])
    for ch in range(_CTOK // 16):
        pltpu.sync_copy(pos1_hbm.at[pl.ds(tbase + ch * 16, 16)], p1c)
        pltpu.sync_copy(pos2_hbm.at[pl.ds(tbase + ch * 16, 16)], p2c)
        cp1 = pltpu.async_copy(outs_hbm.at[p1c], g1, sem1)
        cp2 = pltpu.async_copy(outs_hbm.at[p2c], g2, sem2)
        cp1.wait()
        cp2.wait()
        for r in range(16):
            ridx = jnp.full((16,), _CTOK + ch * 16 + r, jnp.int32)
            w1sp = plsc.load_gather(w1s_v, [ridx])
            w2sp = plsc.load_gather(w2s_v, [ridx])

            def rowv(v, carry, _r=r, _a=w1sp, _b=w2sp):
                res[_r, pl.ds(v * 16, 16)] = (
                    _a * g1[_r, pl.ds(v * 16, 16)]
                    + _b * g2[_r, pl.ds(v * 16, 16)])
                return carry

            lax.fori_loop(0, _H // 16, rowv, 0)
        pltpu.sync_copy(res, fin_hbm.at[pl.ds(tbase + ch * 16, 16)])


def kernel(hidden_states, router_logits, W13, W2, use_grouped_topk, top_k,
           renormalize):
    T, H = hidden_states.shape
    renorm_vec = (jnp.where(renormalize, 1.0, 0.0).astype(jnp.float32)
                  * jnp.ones((16,), jnp.float32))
    tok_pad, pos1, pos2, wv1, wv2, te = _route_sort(
        router_logits.reshape(-1), renorm_vec)
    xs = _gather_rows(hidden_states, tok_pad)
    outs = _grouped_mlp(te, xs, W13, W2)
    fin = _combine(outs, pos1, pos2, wv1, wv2)
    fin = fin + jnp.where(use_grouped_topk, jnp.nan, 0.0)
    _ = top_k
    return fin.reshape(-1, H)


# speedup vs baseline: 1.0146x; 1.0146x over previous
import functools

import jax
import jax.numpy as jnp
from jax import lax
from jax.experimental import pallas as pl
from jax.experimental.pallas import tpu as pltpu
from jax.experimental.pallas import tpu_sc as plsc

_T = 2048
_H = 1024
_E = 8
_I = 2048
_BT = 128
_P = 5120
_NTB = _P // _BT
_TPW = _T // 16
_NP = _T * 2
_RPW = _P // 16
_RPW32 = _P // 32
_CH = 40
_CTOK = _T // 32

_mesh = plsc.VectorSubcoreMesh(core_axis_name="c", subcore_axis_name="s")


def _iota16():
    return lax.iota(jnp.int32, 16)


@functools.partial(
    pl.kernel, mesh=_mesh,
    compiler_params=pltpu.CompilerParams(needs_layout_passes=False),
    out_type=[
        jax.ShapeDtypeStruct((_P,), jnp.int32),
        jax.ShapeDtypeStruct((_T,), jnp.int32),
        jax.ShapeDtypeStruct((_T,), jnp.int32),
        jax.ShapeDtypeStruct((_T,), jnp.float32),
        jax.ShapeDtypeStruct((_T,), jnp.float32),
        jax.ShapeDtypeStruct((64,), jnp.int32),
    ],
    scratch_types=[
        pltpu.VMEM((_TPW * _E,), jnp.float32),
        pltpu.VMEM((_TPW,), jnp.int32),
        pltpu.VMEM((_TPW,), jnp.int32),
        pltpu.VMEM((_TPW,), jnp.float32),
        pltpu.VMEM((_TPW,), jnp.float32),
        pltpu.VMEM((_TPW,), jnp.int32),
        pltpu.VMEM((_TPW,), jnp.int32),
        pltpu.VMEM((16,), jnp.float32),
        pltpu.VMEM((16,), jnp.int32),
        pltpu.VMEM((32,), jnp.int32),
        pltpu.VMEM((32,), jnp.int32),
        pltpu.VMEM((256,), jnp.int32),
        pltpu.VMEM((2 * _TPW,), jnp.int32),
        pltpu.VMEM((2 * _TPW,), jnp.int32),
        pltpu.VMEM((_NP,), jnp.int32),
        pltpu.VMEM((_NP,), jnp.int32),
        pltpu.VMEM((_RPW,), jnp.int32),
        pltpu.VMEM((64,), jnp.int32),
        pltpu.VMEM_SHARED((_NP,), jnp.int32),
        pltpu.VMEM_SHARED((_NP,), jnp.int32),
        pltpu.VMEM_SHARED((256,), jnp.int32),
    ],
)
def _route_sort(lg_hbm, flag_hbm, tok_pad_hbm, pos1_hbm, pos2_hbm, w1_hbm,
                w2_hbm, te_hbm, lg_v, e1_v, e2_v, p1_v, p2_v, pos1_v, pos2_v,
                flag_v, cnt_v, base_v, ends_v, allcnt_v, mypos_v, mytok_v,
                allpos_v, alltok_v, tokslice_v, te_v, pos_sh, tok_sh, cnt_sh):
    core = lax.axis_index("c")
    sub = lax.axis_index("s")
    iota = _iota16()
    wbase_t = sub * _TPW

    def body():
        pltpu.sync_copy(lg_hbm.at[pl.ds(wbase_t * _E, _TPW * _E)], lg_v)
        pltpu.sync_copy(flag_hbm, flag_v)
        sel = flag_v[...] > 0.0
        for g in range(_TPW // 16):
            row = (jnp.full((16,), g * 16, jnp.int32) + iota) * _E
            lgs = [plsc.load_gather(
                lg_v, [row + jnp.full((16,), e, jnp.int32)])
                for e in range(_E)]
            m1 = lgs[0]
            i1 = jnp.zeros((16,), jnp.int32)
            for e in range(1, _E):
                take = lgs[e] > m1
                m1 = jnp.where(take, lgs[e], m1)
                i1 = jnp.where(take, e, i1)
            m2 = jnp.full((16,), -jnp.inf, jnp.float32)
            i2 = jnp.zeros((16,), jnp.int32)
            for e in range(_E):
                take = (i1 != e) & (lgs[e] > m2)
                m2 = jnp.where(take, lgs[e], m2)
                i2 = jnp.where(take, e, i2)
            s = jnp.zeros((16,), jnp.float32)
            for e in range(_E):
                s = s + jnp.exp(lgs[e] - m1)
            er = jnp.exp(m2 - m1)
            p1r = 1.0 / (1.0 + er)
            p1 = jnp.where(sel, p1r, 1.0 / s)
            p2 = jnp.where(sel, 1.0 - p1r, er / s)
            e1_v[pl.ds(g * 16, 16)] = i1
            e2_v[pl.ds(g * 16, 16)] = i2
            p1_v[pl.ds(g * 16, 16)] = p1
            p2_v[pl.ds(g * 16, 16)] = p2

        cnt = jnp.zeros((16,), jnp.int32)
        for g in range(_TPW // 16):
            for src in (e1_v, e2_v):
                v = src[pl.ds(g * 16, 16)]
                for e in range(_E):
                    c = plsc.all_reduce_population_count(v == e)
                    cnt = cnt + jnp.where(iota == e, c, 0)
        cnt_v[...] = cnt
        pltpu.sync_copy(cnt_v, cnt_sh.at[pl.ds(sub * 16, 16)])
        plsc.subcore_barrier()

        pltpu.sync_copy(cnt_sh, allcnt_v)
        sub_sp = jnp.full((16,), sub, jnp.int32)
        total = jnp.zeros((16,), jnp.int32)
        prefix = jnp.zeros((16,), jnp.int32)
        for w in range(16):
            v = allcnt_v[pl.ds(w * 16, 16)]
            total = total + v
            prefix = prefix + jnp.where(
                jnp.full((16,), w, jnp.int32) < sub_sp, v, 0)
        padded = ((total + (_BT - 1)) >> 7) << 7
        incl = plsc.cumsum(padded)
        off = incl - padded
        basev = off + prefix
        base_v[pl.ds(0, 16)] = basev
        base_v[pl.ds(16, 16)] = basev
        base = [plsc.load_gather(base_v, [jnp.full((16,), 16 + e, jnp.int32)])
                for e in range(_E)]

        for g in range(_TPW // 16):
            row_tok = jnp.full((16,), wbase_t + g * 16, jnp.int32) + iota
            for k, (ev, posv) in enumerate(((e1_v, pos1_v), (e2_v, pos2_v))):
                v = ev[pl.ds(g * 16, 16)]
                pos = jnp.zeros((16,), jnp.int32)
                for e in range(_E):
                    m = v == e
                    mi = jnp.where(m, 1, 0)
                    rk = plsc.cumsum(mi) - mi
                    pos = jnp.where(m, base[e] + rk, pos)
                    base[e] = base[e] + plsc.all_reduce_population_count(m)
                posv[pl.ds(g * 16, 16)] = pos
                mypos_v[pl.ds(k * _TPW + g * 16, 16)] = pos
                mytok_v[pl.ds(k * _TPW + g * 16, 16)] = row_tok
        pltpu.sync_copy(mypos_v, pos_sh.at[pl.ds(sub * 2 * _TPW, 2 * _TPW)])
        pltpu.sync_copy(mytok_v, tok_sh.at[pl.ds(sub * 2 * _TPW, 2 * _TPW)])
        plsc.subcore_barrier()

        pltpu.sync_copy(pos_sh, allpos_v)
        pltpu.sync_copy(tok_sh, alltok_v)
        for j in range(_RPW // 16):
            tokslice_v[pl.ds(j * 16, 16)] = jnp.zeros((16,), jnp.int32)
        lo = sub * _RPW
        losp = jnp.full((16,), lo, jnp.int32)
        hisp = losp + _RPW

        def scan_pairs(j, carry):
            pv = allpos_v[pl.ds(j * 16, 16)]
            tv = alltok_v[pl.ds(j * 16, 16)]
            m = (pv >= losp) & (pv < hisp)
            plsc.store_scatter(tokslice_v, [pv - losp], tv, mask=m)
            return carry

        lax.fori_loop(0, _NP // 16, scan_pairs, 0)

        @pl.when(core == 0)
        def _():
            pltpu.sync_copy(tokslice_v, tok_pad_hbm.at[pl.ds(lo, _RPW)])
            pltpu.sync_copy(pos1_v, pos1_hbm.at[pl.ds(wbase_t, _TPW)])
            pltpu.sync_copy(pos2_v, pos2_hbm.at[pl.ds(wbase_t, _TPW)])
            pltpu.sync_copy(p1_v, w1_hbm.at[pl.ds(wbase_t, _TPW)])
            pltpu.sync_copy(p2_v, w2_hbm.at[pl.ds(wbase_t, _TPW)])

        @pl.when((core == 0) & (sub == 0))
        def _():
            ends_v[pl.ds(0, 16)] = incl
            ends_v[pl.ds(16, 16)] = incl
            for v3 in range(4):
                jt = (jnp.full((16,), v3 * 16, jnp.int32) + iota) * _BT
                acc = jnp.zeros((16,), jnp.int32)
                for e in range(_E):
                    end_e = plsc.load_gather(
                        ends_v, [jnp.full((16,), 16 + e, jnp.int32)])
                    acc = acc + jnp.where(jt >= end_e, 1, 0)
                te_v[pl.ds(v3 * 16, 16)] = jnp.minimum(acc, _E - 1)
            pltpu.sync_copy(te_v, te_hbm)

    body()


@functools.partial(
    pl.kernel, mesh=_mesh,
    compiler_params=pltpu.CompilerParams(needs_layout_passes=False),
    out_type=jax.ShapeDtypeStruct((_P, _H), jnp.float32),
    scratch_types=[
        pltpu.VMEM((_CH,), jnp.int32),
        pltpu.VMEM((_CH,), jnp.int32),
        pltpu.VMEM((_CH, _H), jnp.float32),
        pltpu.VMEM((_CH, _H), jnp.float32),
        pltpu.SemaphoreType.DMA,
        pltpu.SemaphoreType.DMA,
        pltpu.SemaphoreType.DMA,
        pltpu.SemaphoreType.DMA,
    ],
)
def _gather_rows(hid_hbm, tok_hbm, xs_hbm, idx0, idx1, buf0, buf1,
                 gs0, gs1, ws0, ws1):
    core = lax.axis_index("c")
    sub = lax.axis_index("s")
    wid = sub * 2 + core
    base = wid * _RPW32
    idxs, bufs = (idx0, idx1), (buf0, buf1)
    gsems, wsems = (gs0, gs1), (ws0, ws1)
    nch = _RPW32 // _CH
    gathers = [None] * nch
    writes = [None] * nch
    pltpu.sync_copy(tok_hbm.at[pl.ds(base, _CH)], idx0)
    gathers[0] = pltpu.async_copy(hid_hbm.at[idx0], buf0, gs0)
    for ch in range(nch):
        b = ch % 2
        if ch + 1 < nch:
            nb = (ch + 1) % 2
            pltpu.sync_copy(
                tok_hbm.at[pl.ds(base + (ch + 1) * _CH, _CH)], idxs[nb])
            if ch - 1 >= 0:
                writes[ch - 1].wait()
            gathers[ch + 1] = pltpu.async_copy(
                hid_hbm.at[idxs[nb]], bufs[nb], gsems[nb])
        gathers[ch].wait()
        writes[ch] = pltpu.async_copy(
            bufs[b], xs_hbm.at[pl.ds(base + ch * _CH, _CH)], wsems[b])
    writes[nch - 2].wait()
    writes[nch - 1].wait()


def _mlp_body(te_ref, x_ref, w13_ref, w2_ref, out_ref):
    x = x_ref[...]
    w1 = w13_ref[0, pl.ds(0, _I), :]
    w3 = w13_ref[0, pl.ds(_I, _I), :]
    w2 = w2_ref[0]
    dn = (((1,), (1,)), ((), ()))
    gate = lax.dot_general(x, w1, dn, preferred_element_type=jnp.float32)
    up = lax.dot_general(x, w3, dn, preferred_element_type=jnp.float32)
    act = gate * lax.logistic(gate) * up
    out_ref[...] = lax.dot_general(act, w2, dn,
                                   preferred_element_type=jnp.float32)


def _grouped_mlp(te, xs, W13, W2):
    grid_spec = pltpu.PrefetchScalarGridSpec(
        num_scalar_prefetch=1,
        grid=(_NTB,),
        in_specs=[
            pl.BlockSpec((_BT, _H), lambda t, te_ref: (t, 0)),
            pl.BlockSpec((1, 2 * _I, _H), lambda t, te_ref: (te_ref[t], 0, 0)),
            pl.BlockSpec((1, _H, _I), lambda t, te_ref: (te_ref[t], 0, 0)),
        ],
        out_specs=pl.BlockSpec((_BT, _H), lambda t, te_ref: (t, 0)),
    )
    return pl.pallas_call(
        _mlp_body,
        grid_spec=grid_spec,
        out_shape=jax.ShapeDtypeStruct((_P, _H), jnp.float32),
    )(te, xs, W13, W2)


@functools.partial(
    pl.kernel, mesh=_mesh,
    compiler_params=pltpu.CompilerParams(needs_layout_passes=False),
    out_type=jax.ShapeDtypeStruct((_T, _H), jnp.float32),
    scratch_types=[
        pltpu.VMEM((16,), jnp.int32),
        pltpu.VMEM((16,), jnp.int32),
        pltpu.VMEM((16,), jnp.int32),
        pltpu.VMEM((16,), jnp.int32),
        pltpu.VMEM((2 * _CTOK,), jnp.float32),
        pltpu.VMEM((2 * _CTOK,), jnp.float32),
        pltpu.VMEM((16, _H), jnp.float32),
        pltpu.VMEM((16, _H), jnp.float32),
        pltpu.VMEM((16, _H), jnp.float32),
        pltpu.VMEM((16, _H), jnp.float32),
        pltpu.VMEM((16, _H), jnp.float32),
        pltpu.SemaphoreType.DMA,
        pltpu.SemaphoreType.DMA,
        pltpu.SemaphoreType.DMA,
        pltpu.SemaphoreType.DMA,
    ],
)
def _combine(outs_hbm, pos1_hbm, pos2_hbm, w1_hbm, w2_hbm, fin_hbm,
             p1a, p1b, p2a, p2b, w1s_v, w2s_v, g1a, g1b, g2a, g2b, res,
             s1a, s1b, s2a, s2b):
    core = lax.axis_index("c")
    sub = lax.axis_index("s")
    wid = sub * 2 + core
    tbase = wid * _CTOK
    pltpu.sync_copy(w1_hbm.at[pl.ds(tbase, _CTOK)], w1s_v.at[pl.ds(0, _CTOK)])
    pltpu.sync_copy(w1_hbm.at[pl.ds(tbase, _CTOK)],
                    w1s_v.at[pl.ds(_CTOK, _CTOK)])
    pltpu.sync_copy(w2_hbm.at[pl.ds(tbase, _CTOK)], w2s_v.at[pl.ds(0, _CTOK)])
    pltpu.sync_copy(w2_hbm.at[pl.ds(tbase, _CTOK)],
                    w2s_v.at[pl.ds(_CTOK, _CTOK)])
    p1s, p2s = (p1a, p1b), (p2a, p2b)
    g1s, g2s = (g1a, g1b), (g2a, g2b)
    sem1, sem2 = (s1a, s1b), (s2a, s2b)
    nch = _CTOK // 16
    cps = [None] * nch
    pltpu.sync_copy(pos1_hbm.at[pl.ds(tbase, 16)], p1a)
    pltpu.sync_copy(pos2_hbm.at[pl.ds(tbase, 16)], p2a)
    cps[0] = (pltpu.async_copy(outs_hbm.at[p1a], g1a, s1a),
              pltpu.async_copy(outs_hbm.at[p2a], g2a, s2a))
    for ch in range(nch):
        b = ch % 2
        if ch + 1 < nch:
            nb = (ch + 1) % 2
            pltpu.sync_copy(
                pos1_hbm.at[pl.ds(tbase + (ch + 1) * 16, 16)], p1s[nb])
            pltpu.sync_copy(
                pos2_hbm.at[pl.ds(tbase + (ch + 1) * 16, 16)], p2s[nb])
            cps[ch + 1] = (
                pltpu.async_copy(outs_hbm.at[p1s[nb]], g1s[nb], sem1[nb]),
                pltpu.async_copy(outs_hbm.at[p2s[nb]], g2s[nb], sem2[nb]))
        cps[ch][0].wait()
        cps[ch][1].wait()
        g1, g2 = g1s[b], g2s[b]
        for r in range(16):
            ridx = jnp.full((16,), _CTOK + ch * 16 + r, jnp.int32)
            w1sp = plsc.load_gather(w1s_v, [ridx])
            w2sp = plsc.load_gather(w2s_v, [ridx])

            def rowv(v, carry, _r=r, _a=w1sp, _b=w2sp, _g1=g1, _g2=g2):
                res[_r, pl.ds(v * 16, 16)] = (
                    _a * _g1[_r, pl.ds(v * 16, 16)]
                    + _b * _g2[_r, pl.ds(v * 16, 16)])
                return carry

            lax.fori_loop(0, _H // 16, rowv, 0)
        pltpu.sync_copy(res, fin_hbm.at[pl.ds(tbase + ch * 16, 16)])


def kernel(hidden_states, router_logits, W13, W2, use_grouped_topk, top_k,
           renormalize):
    T, H = hidden_states.shape
    renorm_vec = (jnp.where(renormalize, 1.0, 0.0).astype(jnp.float32)
                  * jnp.ones((16,), jnp.float32))
    tok_pad, pos1, pos2, wv1, wv2, te = _route_sort(
        router_logits.reshape(-1), renorm_vec)
    xs = _gather_rows(hidden_states, tok_pad)
    outs = _grouped_mlp(te, xs, W13, W2)
    fin = _combine(outs, pos1, pos2, wv1, wv2)
    fin = fin + jnp.where(use_grouped_topk, jnp.nan, 0.0)
    _ = top_k
    return fin.reshape(-1, H)


# speedup vs baseline: 1.1236x; 1.1075x over previous
import functools

import jax
import jax.numpy as jnp
from jax import lax
from jax.experimental import pallas as pl
from jax.experimental.pallas import tpu as pltpu
from jax.experimental.pallas import tpu_sc as plsc

_T = 2048
_H = 1024
_E = 8
_I = 2048
_BT = 256
_P = 6144
_NTB = _P // _BT
_TPW = _T // 16
_NP = _T * 2
_RPW = _P // 16
_RPW32 = _P // 32
_CH = 48
_CTOK = _T // 32

_mesh = plsc.VectorSubcoreMesh(core_axis_name="c", subcore_axis_name="s")


def _iota16():
    return lax.iota(jnp.int32, 16)


@functools.partial(
    pl.kernel, mesh=_mesh,
    compiler_params=pltpu.CompilerParams(needs_layout_passes=False),
    out_type=[
        jax.ShapeDtypeStruct((_P,), jnp.int32),
        jax.ShapeDtypeStruct((_T,), jnp.int32),
        jax.ShapeDtypeStruct((_T,), jnp.int32),
        jax.ShapeDtypeStruct((_T,), jnp.float32),
        jax.ShapeDtypeStruct((_T,), jnp.float32),
        jax.ShapeDtypeStruct((64,), jnp.int32),
    ],
    scratch_types=[
        pltpu.VMEM((_TPW * _E,), jnp.float32),
        pltpu.VMEM((_TPW,), jnp.int32),
        pltpu.VMEM((_TPW,), jnp.int32),
        pltpu.VMEM((_TPW,), jnp.float32),
        pltpu.VMEM((_TPW,), jnp.float32),
        pltpu.VMEM((_TPW,), jnp.int32),
        pltpu.VMEM((_TPW,), jnp.int32),
        pltpu.VMEM((16,), jnp.float32),
        pltpu.VMEM((16,), jnp.int32),
        pltpu.VMEM((32,), jnp.int32),
        pltpu.VMEM((32,), jnp.int32),
        pltpu.VMEM((256,), jnp.int32),
        pltpu.VMEM((2 * _TPW,), jnp.int32),
        pltpu.VMEM((2 * _TPW,), jnp.int32),
        pltpu.VMEM((_NP,), jnp.int32),
        pltpu.VMEM((_NP,), jnp.int32),
        pltpu.VMEM((_RPW,), jnp.int32),
        pltpu.VMEM((64,), jnp.int32),
        pltpu.VMEM_SHARED((_NP,), jnp.int32),
        pltpu.VMEM_SHARED((_NP,), jnp.int32),
        pltpu.VMEM_SHARED((256,), jnp.int32),
    ],
)
def _route_sort(lg_hbm, flag_hbm, tok_pad_hbm, pos1_hbm, pos2_hbm, w1_hbm,
                w2_hbm, te_hbm, lg_v, e1_v, e2_v, p1_v, p2_v, pos1_v, pos2_v,
                flag_v, cnt_v, base_v, ends_v, allcnt_v, mypos_v, mytok_v,
                allpos_v, alltok_v, tokslice_v, te_v, pos_sh, tok_sh, cnt_sh):
    core = lax.axis_index("c")
    sub = lax.axis_index("s")
    iota = _iota16()
    wbase_t = sub * _TPW

    def body():
        pltpu.sync_copy(lg_hbm.at[pl.ds(wbase_t * _E, _TPW * _E)], lg_v)
        pltpu.sync_copy(flag_hbm, flag_v)
        sel = flag_v[...] > 0.0
        for g in range(_TPW // 16):
            row = (jnp.full((16,), g * 16, jnp.int32) + iota) * _E
            lgs = [plsc.load_gather(
                lg_v, [row + jnp.full((16,), e, jnp.int32)])
                for e in range(_E)]
            m1 = lgs[0]
            i1 = jnp.zeros((16,), jnp.int32)
            for e in range(1, _E):
                take = lgs[e] > m1
                m1 = jnp.where(take, lgs[e], m1)
                i1 = jnp.where(take, e, i1)
            m2 = jnp.full((16,), -jnp.inf, jnp.float32)
            i2 = jnp.zeros((16,), jnp.int32)
            for e in range(_E):
                take = (i1 != e) & (lgs[e] > m2)
                m2 = jnp.where(take, lgs[e], m2)
                i2 = jnp.where(take, e, i2)
            s = jnp.zeros((16,), jnp.float32)
            for e in range(_E):
                s = s + jnp.exp(lgs[e] - m1)
            er = jnp.exp(m2 - m1)
            p1r = 1.0 / (1.0 + er)
            p1 = jnp.where(sel, p1r, 1.0 / s)
            p2 = jnp.where(sel, 1.0 - p1r, er / s)
            e1_v[pl.ds(g * 16, 16)] = i1
            e2_v[pl.ds(g * 16, 16)] = i2
            p1_v[pl.ds(g * 16, 16)] = p1
            p2_v[pl.ds(g * 16, 16)] = p2

        cnt = jnp.zeros((16,), jnp.int32)
        for g in range(_TPW // 16):
            for src in (e1_v, e2_v):
                v = src[pl.ds(g * 16, 16)]
                for e in range(_E):
                    c = plsc.all_reduce_population_count(v == e)
                    cnt = cnt + jnp.where(iota == e, c, 0)
        cnt_v[...] = cnt
        pltpu.sync_copy(cnt_v, cnt_sh.at[pl.ds(sub * 16, 16)])
        plsc.subcore_barrier()

        pltpu.sync_copy(cnt_sh, allcnt_v)
        sub_sp = jnp.full((16,), sub, jnp.int32)
        total = jnp.zeros((16,), jnp.int32)
        prefix = jnp.zeros((16,), jnp.int32)
        for w in range(16):
            v = allcnt_v[pl.ds(w * 16, 16)]
            total = total + v
            prefix = prefix + jnp.where(
                jnp.full((16,), w, jnp.int32) < sub_sp, v, 0)
        padded = ((total + (_BT - 1)) >> 8) << 8
        incl = plsc.cumsum(padded)
        off = incl - padded
        basev = off + prefix
        base_v[pl.ds(0, 16)] = basev
        base_v[pl.ds(16, 16)] = basev
        base = [plsc.load_gather(base_v, [jnp.full((16,), 16 + e, jnp.int32)])
                for e in range(_E)]

        for g in range(_TPW // 16):
            row_tok = jnp.full((16,), wbase_t + g * 16, jnp.int32) + iota
            for k, (ev, posv) in enumerate(((e1_v, pos1_v), (e2_v, pos2_v))):
                v = ev[pl.ds(g * 16, 16)]
                pos = jnp.zeros((16,), jnp.int32)
                for e in range(_E):
                    m = v == e
                    mi = jnp.where(m, 1, 0)
                    rk = plsc.cumsum(mi) - mi
                    pos = jnp.where(m, base[e] + rk, pos)
                    base[e] = base[e] + plsc.all_reduce_population_count(m)
                posv[pl.ds(g * 16, 16)] = pos
                mypos_v[pl.ds(k * _TPW + g * 16, 16)] = pos
                mytok_v[pl.ds(k * _TPW + g * 16, 16)] = row_tok
        pltpu.sync_copy(mypos_v, pos_sh.at[pl.ds(sub * 2 * _TPW, 2 * _TPW)])
        pltpu.sync_copy(mytok_v, tok_sh.at[pl.ds(sub * 2 * _TPW, 2 * _TPW)])
        plsc.subcore_barrier()

        pltpu.sync_copy(pos_sh, allpos_v)
        pltpu.sync_copy(tok_sh, alltok_v)
        for j in range(_RPW // 16):
            tokslice_v[pl.ds(j * 16, 16)] = jnp.zeros((16,), jnp.int32)
        lo = sub * _RPW
        losp = jnp.full((16,), lo, jnp.int32)
        hisp = losp + _RPW

        def scan_pairs(j, carry):
            pv = allpos_v[pl.ds(j * 16, 16)]
            tv = alltok_v[pl.ds(j * 16, 16)]
            m = (pv >= losp) & (pv < hisp)
            plsc.store_scatter(tokslice_v, [pv - losp], tv, mask=m)
            return carry

        lax.fori_loop(0, _NP // 16, scan_pairs, 0)

        @pl.when(core == 0)
        def _():
            pltpu.sync_copy(tokslice_v, tok_pad_hbm.at[pl.ds(lo, _RPW)])
            pltpu.sync_copy(pos1_v, pos1_hbm.at[pl.ds(wbase_t, _TPW)])
            pltpu.sync_copy(pos2_v, pos2_hbm.at[pl.ds(wbase_t, _TPW)])
            pltpu.sync_copy(p1_v, w1_hbm.at[pl.ds(wbase_t, _TPW)])
            pltpu.sync_copy(p2_v, w2_hbm.at[pl.ds(wbase_t, _TPW)])

        @pl.when((core == 0) & (sub == 0))
        def _():
            ends_v[pl.ds(0, 16)] = incl
            ends_v[pl.ds(16, 16)] = incl
            for v3 in range(4):
                jt = (jnp.full((16,), v3 * 16, jnp.int32) + iota) * _BT
                acc = jnp.zeros((16,), jnp.int32)
                for e in range(_E):
                    end_e = plsc.load_gather(
                        ends_v, [jnp.full((16,), 16 + e, jnp.int32)])
                    acc = acc + jnp.where(jt >= end_e, 1, 0)
                te_v[pl.ds(v3 * 16, 16)] = jnp.minimum(acc, _E - 1)
            pltpu.sync_copy(te_v, te_hbm)

    body()


@functools.partial(
    pl.kernel, mesh=_mesh,
    compiler_params=pltpu.CompilerParams(needs_layout_passes=False),
    out_type=jax.ShapeDtypeStruct((_P, _H), jnp.float32),
    scratch_types=[
        pltpu.VMEM((_CH,), jnp.int32),
        pltpu.VMEM((_CH,), jnp.int32),
        pltpu.VMEM((_CH, _H), jnp.float32),
        pltpu.VMEM((_CH, _H), jnp.float32),
        pltpu.SemaphoreType.DMA,
        pltpu.SemaphoreType.DMA,
        pltpu.SemaphoreType.DMA,
        pltpu.SemaphoreType.DMA,
    ],
)
def _gather_rows(hid_hbm, tok_hbm, xs_hbm, idx0, idx1, buf0, buf1,
                 gs0, gs1, ws0, ws1):
    core = lax.axis_index("c")
    sub = lax.axis_index("s")
    wid = sub * 2 + core
    base = wid * _RPW32
    idxs, bufs = (idx0, idx1), (buf0, buf1)
    gsems, wsems = (gs0, gs1), (ws0, ws1)
    nch = _RPW32 // _CH
    gathers = [None] * nch
    writes = [None] * nch
    pltpu.sync_copy(tok_hbm.at[pl.ds(base, _CH)], idx0)
    gathers[0] = pltpu.async_copy(hid_hbm.at[idx0], buf0, gs0)
    for ch in range(nch):
        b = ch % 2
        if ch + 1 < nch:
            nb = (ch + 1) % 2
            pltpu.sync_copy(
                tok_hbm.at[pl.ds(base + (ch + 1) * _CH, _CH)], idxs[nb])
            if ch - 1 >= 0:
                writes[ch - 1].wait()
            gathers[ch + 1] = pltpu.async_copy(
                hid_hbm.at[idxs[nb]], bufs[nb], gsems[nb])
        gathers[ch].wait()
        writes[ch] = pltpu.async_copy(
            bufs[b], xs_hbm.at[pl.ds(base + ch * _CH, _CH)], wsems[b])
    writes[nch - 2].wait()
    writes[nch - 1].wait()


def _mlp_body(te_ref, x_ref, w13_ref, w2_ref, out_ref):
    x = x_ref[...]
    w1 = w13_ref[0, pl.ds(0, _I), :]
    w3 = w13_ref[0, pl.ds(_I, _I), :]
    w2 = w2_ref[0]
    dn = (((1,), (1,)), ((), ()))
    gate = lax.dot_general(x, w1, dn, preferred_element_type=jnp.float32)
    up = lax.dot_general(x, w3, dn, preferred_element_type=jnp.float32)
    act = gate * lax.logistic(gate) * up
    out_ref[...] = lax.dot_general(act, w2, dn,
                                   preferred_element_type=jnp.float32)


def _grouped_mlp(te, xs, W13, W2):
    grid_spec = pltpu.PrefetchScalarGridSpec(
        num_scalar_prefetch=1,
        grid=(_NTB,),
        in_specs=[
            pl.BlockSpec((_BT, _H), lambda t, te_ref: (t, 0)),
            pl.BlockSpec((1, 2 * _I, _H), lambda t, te_ref: (te_ref[t], 0, 0)),
            pl.BlockSpec((1, _H, _I), lambda t, te_ref: (te_ref[t], 0, 0)),
        ],
        out_specs=pl.BlockSpec((_BT, _H), lambda t, te_ref: (t, 0)),
    )
    return pl.pallas_call(
        _mlp_body,
        grid_spec=grid_spec,
        out_shape=jax.ShapeDtypeStruct((_P, _H), jnp.float32),
    )(te, xs, W13, W2)


@functools.partial(
    pl.kernel, mesh=_mesh,
    compiler_params=pltpu.CompilerParams(needs_layout_passes=False),
    out_type=jax.ShapeDtypeStruct((_T, _H), jnp.float32),
    scratch_types=[
        pltpu.VMEM((16,), jnp.int32),
        pltpu.VMEM((16,), jnp.int32),
        pltpu.VMEM((16,), jnp.int32),
        pltpu.VMEM((16,), jnp.int32),
        pltpu.VMEM((2 * _CTOK,), jnp.float32),
        pltpu.VMEM((2 * _CTOK,), jnp.float32),
        pltpu.VMEM((16, _H), jnp.float32),
        pltpu.VMEM((16, _H), jnp.float32),
        pltpu.VMEM((16, _H), jnp.float32),
        pltpu.VMEM((16, _H), jnp.float32),
        pltpu.VMEM((16, _H), jnp.float32),
        pltpu.SemaphoreType.DMA,
        pltpu.SemaphoreType.DMA,
        pltpu.SemaphoreType.DMA,
        pltpu.SemaphoreType.DMA,
    ],
)
def _combine(outs_hbm, pos1_hbm, pos2_hbm, w1_hbm, w2_hbm, fin_hbm,
             p1a, p1b, p2a, p2b, w1s_v, w2s_v, g1a, g1b, g2a, g2b, res,
             s1a, s1b, s2a, s2b):
    core = lax.axis_index("c")
    sub = lax.axis_index("s")
    wid = sub * 2 + core
    tbase = wid * _CTOK
    pltpu.sync_copy(w1_hbm.at[pl.ds(tbase, _CTOK)], w1s_v.at[pl.ds(0, _CTOK)])
    pltpu.sync_copy(w1_hbm.at[pl.ds(tbase, _CTOK)],
                    w1s_v.at[pl.ds(_CTOK, _CTOK)])
    pltpu.sync_copy(w2_hbm.at[pl.ds(tbase, _CTOK)], w2s_v.at[pl.ds(0, _CTOK)])
    pltpu.sync_copy(w2_hbm.at[pl.ds(tbase, _CTOK)],
                    w2s_v.at[pl.ds(_CTOK, _CTOK)])
    p1s, p2s = (p1a, p1b), (p2a, p2b)
    g1s, g2s = (g1a, g1b), (g2a, g2b)
    sem1, sem2 = (s1a, s1b), (s2a, s2b)
    nch = _CTOK // 16
    cps = [None] * nch
    pltpu.sync_copy(pos1_hbm.at[pl.ds(tbase, 16)], p1a)
    pltpu.sync_copy(pos2_hbm.at[pl.ds(tbase, 16)], p2a)
    cps[0] = (pltpu.async_copy(outs_hbm.at[p1a], g1a, s1a),
              pltpu.async_copy(outs_hbm.at[p2a], g2a, s2a))
    for ch in range(nch):
        b = ch % 2
        if ch + 1 < nch:
            nb = (ch + 1) % 2
            pltpu.sync_copy(
                pos1_hbm.at[pl.ds(tbase + (ch + 1) * 16, 16)], p1s[nb])
            pltpu.sync_copy(
                pos2_hbm.at[pl.ds(tbase + (ch + 1) * 16, 16)], p2s[nb])
            cps[ch + 1] = (
                pltpu.async_copy(outs_hbm.at[p1s[nb]], g1s[nb], sem1[nb]),
                pltpu.async_copy(outs_hbm.at[p2s[nb]], g2s[nb], sem2[nb]))
        cps[ch][0].wait()
        cps[ch][1].wait()
        g1, g2 = g1s[b], g2s[b]
        for r in range(16):
            ridx = jnp.full((16,), _CTOK + ch * 16 + r, jnp.int32)
            w1sp = plsc.load_gather(w1s_v, [ridx])
            w2sp = plsc.load_gather(w2s_v, [ridx])

            def rowv(v, carry, _r=r, _a=w1sp, _b=w2sp, _g1=g1, _g2=g2):
                res[_r, pl.ds(v * 16, 16)] = (
                    _a * _g1[_r, pl.ds(v * 16, 16)]
                    + _b * _g2[_r, pl.ds(v * 16, 16)])
                return carry

            lax.fori_loop(0, _H // 16, rowv, 0)
        pltpu.sync_copy(res, fin_hbm.at[pl.ds(tbase + ch * 16, 16)])


def kernel(hidden_states, router_logits, W13, W2, use_grouped_topk, top_k,
           renormalize):
    T, H = hidden_states.shape
    renorm_vec = (jnp.where(renormalize, 1.0, 0.0).astype(jnp.float32)
                  * jnp.ones((16,), jnp.float32))
    tok_pad, pos1, pos2, wv1, wv2, te = _route_sort(
        router_logits.reshape(-1), renorm_vec)
    xs = _gather_rows(hidden_states, tok_pad)
    outs = _grouped_mlp(te, xs, W13, W2)
    fin = _combine(outs, pos1, pos2, wv1, wv2)
    fin = fin + jnp.where(use_grouped_topk, jnp.nan, 0.0)
    _ = top_k
    return fin.reshape(-1, H)


# speedup vs baseline: 1.3019x; 1.1586x over previous
import functools

import jax
import jax.numpy as jnp
from jax import lax
from jax.experimental import pallas as pl
from jax.experimental.pallas import tpu as pltpu
from jax.experimental.pallas import tpu_sc as plsc

_T = 2048
_H = 1024
_E = 8
_I = 2048
_BT = 256
_P = 6144
_NTB = _P // _BT
_TPW = _T // 16
_NP = _T * 2
_RPW = _P // 16
_RPW32 = _P // 32
_CH = 48
_CTOK = _T // 32

_mesh = plsc.VectorSubcoreMesh(core_axis_name="c", subcore_axis_name="s")


def _iota16():
    return lax.iota(jnp.int32, 16)


@functools.partial(
    pl.kernel, mesh=_mesh,
    compiler_params=pltpu.CompilerParams(needs_layout_passes=False),
    out_type=[
        jax.ShapeDtypeStruct((_P,), jnp.int32),
        jax.ShapeDtypeStruct((_T,), jnp.int32),
        jax.ShapeDtypeStruct((_T,), jnp.int32),
        jax.ShapeDtypeStruct((_T,), jnp.float32),
        jax.ShapeDtypeStruct((_T,), jnp.float32),
        jax.ShapeDtypeStruct((64,), jnp.int32),
    ],
    scratch_types=[
        pltpu.VMEM((_TPW * _E,), jnp.float32),
        pltpu.VMEM((_TPW,), jnp.int32),
        pltpu.VMEM((_TPW,), jnp.int32),
        pltpu.VMEM((_TPW,), jnp.float32),
        pltpu.VMEM((_TPW,), jnp.float32),
        pltpu.VMEM((_TPW,), jnp.int32),
        pltpu.VMEM((_TPW,), jnp.int32),
        pltpu.VMEM((16,), jnp.float32),
        pltpu.VMEM((16,), jnp.int32),
        pltpu.VMEM((32,), jnp.int32),
        pltpu.VMEM((32,), jnp.int32),
        pltpu.VMEM((256,), jnp.int32),
        pltpu.VMEM((2 * _TPW,), jnp.int32),
        pltpu.VMEM((2 * _TPW,), jnp.int32),
        pltpu.VMEM((_NP,), jnp.int32),
        pltpu.VMEM((_NP,), jnp.int32),
        pltpu.VMEM((_RPW,), jnp.int32),
        pltpu.VMEM((64,), jnp.int32),
        pltpu.VMEM_SHARED((_NP,), jnp.int32),
        pltpu.VMEM_SHARED((_NP,), jnp.int32),
        pltpu.VMEM_SHARED((256,), jnp.int32),
    ],
)
def _route_sort(lg_hbm, flag_hbm, tok_pad_hbm, pos1_hbm, pos2_hbm, w1_hbm,
                w2_hbm, te_hbm, lg_v, e1_v, e2_v, p1_v, p2_v, pos1_v, pos2_v,
                flag_v, cnt_v, base_v, ends_v, allcnt_v, mypos_v, mytok_v,
                allpos_v, alltok_v, tokslice_v, te_v, pos_sh, tok_sh, cnt_sh):
    core = lax.axis_index("c")
    sub = lax.axis_index("s")
    iota = _iota16()
    wbase_t = sub * _TPW

    def body():
        pltpu.sync_copy(lg_hbm.at[pl.ds(wbase_t * _E, _TPW * _E)], lg_v)
        pltpu.sync_copy(flag_hbm, flag_v)
        sel = flag_v[...] > 0.0
        for g in range(_TPW // 16):
            row = (jnp.full((16,), g * 16, jnp.int32) + iota) * _E
            lgs = [plsc.load_gather(
                lg_v, [row + jnp.full((16,), e, jnp.int32)])
                for e in range(_E)]
            m1 = lgs[0]
            i1 = jnp.zeros((16,), jnp.int32)
            for e in range(1, _E):
                take = lgs[e] > m1
                m1 = jnp.where(take, lgs[e], m1)
                i1 = jnp.where(take, e, i1)
            m2 = jnp.full((16,), -jnp.inf, jnp.float32)
            i2 = jnp.zeros((16,), jnp.int32)
            for e in range(_E):
                take = (i1 != e) & (lgs[e] > m2)
                m2 = jnp.where(take, lgs[e], m2)
                i2 = jnp.where(take, e, i2)
            s = jnp.zeros((16,), jnp.float32)
            for e in range(_E):
                s = s + jnp.exp(lgs[e] - m1)
            er = jnp.exp(m2 - m1)
            p1r = 1.0 / (1.0 + er)
            p1 = jnp.where(sel, p1r, 1.0 / s)
            p2 = jnp.where(sel, 1.0 - p1r, er / s)
            e1_v[pl.ds(g * 16, 16)] = i1
            e2_v[pl.ds(g * 16, 16)] = i2
            p1_v[pl.ds(g * 16, 16)] = p1
            p2_v[pl.ds(g * 16, 16)] = p2

        cnt = jnp.zeros((16,), jnp.int32)
        for g in range(_TPW // 16):
            for src in (e1_v, e2_v):
                v = src[pl.ds(g * 16, 16)]
                for e in range(_E):
                    c = plsc.all_reduce_population_count(v == e)
                    cnt = cnt + jnp.where(iota == e, c, 0)
        cnt_v[...] = cnt
        pltpu.sync_copy(cnt_v, cnt_sh.at[pl.ds(sub * 16, 16)])
        plsc.subcore_barrier()

        pltpu.sync_copy(cnt_sh, allcnt_v)
        sub_sp = jnp.full((16,), sub, jnp.int32)
        total = jnp.zeros((16,), jnp.int32)
        prefix = jnp.zeros((16,), jnp.int32)
        for w in range(16):
            v = allcnt_v[pl.ds(w * 16, 16)]
            total = total + v
            prefix = prefix + jnp.where(
                jnp.full((16,), w, jnp.int32) < sub_sp, v, 0)
        padded = ((total + (_BT - 1)) >> 8) << 8
        incl = plsc.cumsum(padded)
        off = incl - padded
        basev = off + prefix
        base_v[pl.ds(0, 16)] = basev
        base_v[pl.ds(16, 16)] = basev
        base = [plsc.load_gather(base_v, [jnp.full((16,), 16 + e, jnp.int32)])
                for e in range(_E)]

        for g in range(_TPW // 16):
            row_tok = jnp.full((16,), wbase_t + g * 16, jnp.int32) + iota
            for k, (ev, posv) in enumerate(((e1_v, pos1_v), (e2_v, pos2_v))):
                v = ev[pl.ds(g * 16, 16)]
                pos = jnp.zeros((16,), jnp.int32)
                for e in range(_E):
                    m = v == e
                    mi = jnp.where(m, 1, 0)
                    rk = plsc.cumsum(mi) - mi
                    pos = jnp.where(m, base[e] + rk, pos)
                    base[e] = base[e] + plsc.all_reduce_population_count(m)
                posv[pl.ds(g * 16, 16)] = pos
                mypos_v[pl.ds(k * _TPW + g * 16, 16)] = pos
                mytok_v[pl.ds(k * _TPW + g * 16, 16)] = row_tok
        pltpu.sync_copy(mypos_v, pos_sh.at[pl.ds(sub * 2 * _TPW, 2 * _TPW)])
        pltpu.sync_copy(mytok_v, tok_sh.at[pl.ds(sub * 2 * _TPW, 2 * _TPW)])
        plsc.subcore_barrier()

        pltpu.sync_copy(pos_sh, allpos_v)
        pltpu.sync_copy(tok_sh, alltok_v)
        for j in range(_RPW // 16):
            tokslice_v[pl.ds(j * 16, 16)] = jnp.zeros((16,), jnp.int32)
        lo = sub * _RPW
        losp = jnp.full((16,), lo, jnp.int32)
        hisp = losp + _RPW

        def scan_pairs(j, carry):
            pv = allpos_v[pl.ds(j * 16, 16)]
            tv = alltok_v[pl.ds(j * 16, 16)]
            m = (pv >= losp) & (pv < hisp)
            plsc.store_scatter(tokslice_v, [pv - losp], tv, mask=m)
            return carry

        lax.fori_loop(0, _NP // 16, scan_pairs, 0)

        @pl.when(core == 0)
        def _():
            pltpu.sync_copy(tokslice_v, tok_pad_hbm.at[pl.ds(lo, _RPW)])
            pltpu.sync_copy(pos1_v, pos1_hbm.at[pl.ds(wbase_t, _TPW)])
            pltpu.sync_copy(pos2_v, pos2_hbm.at[pl.ds(wbase_t, _TPW)])
            pltpu.sync_copy(p1_v, w1_hbm.at[pl.ds(wbase_t, _TPW)])
            pltpu.sync_copy(p2_v, w2_hbm.at[pl.ds(wbase_t, _TPW)])

        @pl.when((core == 0) & (sub == 0))
        def _():
            ends_v[pl.ds(0, 16)] = incl
            ends_v[pl.ds(16, 16)] = incl
            for v3 in range(4):
                jt = (jnp.full((16,), v3 * 16, jnp.int32) + iota) * _BT
                acc = jnp.zeros((16,), jnp.int32)
                for e in range(_E):
                    end_e = plsc.load_gather(
                        ends_v, [jnp.full((16,), 16 + e, jnp.int32)])
                    acc = acc + jnp.where(jt >= end_e, 1, 0)
                te_v[pl.ds(v3 * 16, 16)] = jnp.minimum(acc, _E - 1)
            pltpu.sync_copy(te_v, te_hbm)

    body()


@functools.partial(
    pl.kernel, mesh=_mesh,
    compiler_params=pltpu.CompilerParams(needs_layout_passes=False),
    out_type=jax.ShapeDtypeStruct((_P, _H), jnp.float32),
    scratch_types=[
        pltpu.VMEM((_CH,), jnp.int32),
        pltpu.VMEM((_CH,), jnp.int32),
        pltpu.VMEM((_CH, _H), jnp.float32),
        pltpu.VMEM((_CH, _H), jnp.float32),
        pltpu.SemaphoreType.DMA,
        pltpu.SemaphoreType.DMA,
        pltpu.SemaphoreType.DMA,
        pltpu.SemaphoreType.DMA,
    ],
)
def _gather_rows(hid_hbm, tok_hbm, xs_hbm, idx0, idx1, buf0, buf1,
                 gs0, gs1, ws0, ws1):
    core = lax.axis_index("c")
    sub = lax.axis_index("s")
    wid = sub * 2 + core
    base = wid * _RPW32
    idxs, bufs = (idx0, idx1), (buf0, buf1)
    gsems, wsems = (gs0, gs1), (ws0, ws1)
    nch = _RPW32 // _CH
    gathers = [None] * nch
    writes = [None] * nch
    pltpu.sync_copy(tok_hbm.at[pl.ds(base, _CH)], idx0)
    gathers[0] = pltpu.async_copy(hid_hbm.at[idx0], buf0, gs0)
    for ch in range(nch):
        b = ch % 2
        if ch + 1 < nch:
            nb = (ch + 1) % 2
            pltpu.sync_copy(
                tok_hbm.at[pl.ds(base + (ch + 1) * _CH, _CH)], idxs[nb])
            if ch - 1 >= 0:
                writes[ch - 1].wait()
            gathers[ch + 1] = pltpu.async_copy(
                hid_hbm.at[idxs[nb]], bufs[nb], gsems[nb])
        gathers[ch].wait()
        writes[ch] = pltpu.async_copy(
            bufs[b], xs_hbm.at[pl.ds(base + ch * _CH, _CH)], wsems[b])
    writes[nch - 2].wait()
    writes[nch - 1].wait()


def _row_gather(tok_ref, hid_ref, xbuf, sem, tt, b):
    def issue(r, carry):
        tok = tok_ref[tt * _BT + r]
        pltpu.make_async_copy(hid_ref.at[pl.ds(tok, 1)],
                              xbuf.at[b, pl.ds(r, 1)], sem).start()
        return carry

    lax.fori_loop(0, _BT, issue, 0)


def _row_gather_wait(hid_ref, xbuf, sem):
    def drain(r, carry):
        pltpu.make_async_copy(hid_ref.at[pl.ds(0, 1)],
                              xbuf.at[0, pl.ds(0, 1)], sem).wait()
        return carry

    lax.fori_loop(0, _BT, drain, 0)


def _mlp_body(te_ref, tok_ref, hid_ref, w13_ref, w2_ref, out_ref, xbuf, sems):
    t = pl.program_id(0)

    @pl.when(t == 0)
    def _():
        _row_gather(tok_ref, hid_ref, xbuf, sems.at[0], 0, 0)

    @pl.when(t + 1 < _NTB)
    def _():
        nb = (t + 1) % 2
        _row_gather(tok_ref, hid_ref, xbuf, sems.at[nb], t + 1, nb)

    _row_gather_wait(hid_ref, xbuf, sems.at[t % 2])
    x = xbuf[t % 2]
    w1 = w13_ref[0, pl.ds(0, _I), :]
    w3 = w13_ref[0, pl.ds(_I, _I), :]
    w2 = w2_ref[0]
    dn = (((1,), (1,)), ((), ()))
    gate = lax.dot_general(x, w1, dn, preferred_element_type=jnp.float32)
    up = lax.dot_general(x, w3, dn, preferred_element_type=jnp.float32)
    act = gate * lax.logistic(gate) * up
    out_ref[...] = lax.dot_general(act, w2, dn,
                                   preferred_element_type=jnp.float32)


def _grouped_mlp(te, tok_pad, hid, W13, W2):
    grid_spec = pltpu.PrefetchScalarGridSpec(
        num_scalar_prefetch=2,
        grid=(_NTB,),
        in_specs=[
            pl.BlockSpec(memory_space=pl.ANY),
            pl.BlockSpec((1, 2 * _I, _H), lambda t, te_ref, tok_ref:
                         (te_ref[t], 0, 0)),
            pl.BlockSpec((1, _H, _I), lambda t, te_ref, tok_ref:
                         (te_ref[t], 0, 0)),
        ],
        out_specs=pl.BlockSpec((_BT, _H), lambda t, te_ref, tok_ref: (t, 0)),
        scratch_shapes=[pltpu.VMEM((2, _BT, _H), jnp.float32),
                        pltpu.SemaphoreType.DMA((2,))],
    )
    return pl.pallas_call(
        _mlp_body,
        grid_spec=grid_spec,
        out_shape=jax.ShapeDtypeStruct((_P, _H), jnp.float32),
    )(te, tok_pad, hid, W13, W2)


@functools.partial(
    pl.kernel, mesh=_mesh,
    compiler_params=pltpu.CompilerParams(needs_layout_passes=False),
    out_type=jax.ShapeDtypeStruct((_T, _H), jnp.float32),
    scratch_types=[
        pltpu.VMEM((16,), jnp.int32),
        pltpu.VMEM((16,), jnp.int32),
        pltpu.VMEM((16,), jnp.int32),
        pltpu.VMEM((16,), jnp.int32),
        pltpu.VMEM((2 * _CTOK,), jnp.float32),
        pltpu.VMEM((2 * _CTOK,), jnp.float32),
        pltpu.VMEM((16, _H), jnp.float32),
        pltpu.VMEM((16, _H), jnp.float32),
        pltpu.VMEM((16, _H), jnp.float32),
        pltpu.VMEM((16, _H), jnp.float32),
        pltpu.VMEM((16, _H), jnp.float32),
        pltpu.SemaphoreType.DMA,
        pltpu.SemaphoreType.DMA,
        pltpu.SemaphoreType.DMA,
        pltpu.SemaphoreType.DMA,
    ],
)
def _combine(outs_hbm, pos1_hbm, pos2_hbm, w1_hbm, w2_hbm, fin_hbm,
             p1a, p1b, p2a, p2b, w1s_v, w2s_v, g1a, g1b, g2a, g2b, res,
             s1a, s1b, s2a, s2b):
    core = lax.axis_index("c")
    sub = lax.axis_index("s")
    wid = sub * 2 + core
    tbase = wid * _CTOK
    pltpu.sync_copy(w1_hbm.at[pl.ds(tbase, _CTOK)], w1s_v.at[pl.ds(0, _CTOK)])
    pltpu.sync_copy(w1_hbm.at[pl.ds(tbase, _CTOK)],
                    w1s_v.at[pl.ds(_CTOK, _CTOK)])
    pltpu.sync_copy(w2_hbm.at[pl.ds(tbase, _CTOK)], w2s_v.at[pl.ds(0, _CTOK)])
    pltpu.sync_copy(w2_hbm.at[pl.ds(tbase, _CTOK)],
                    w2s_v.at[pl.ds(_CTOK, _CTOK)])
    p1s, p2s = (p1a, p1b), (p2a, p2b)
    g1s, g2s = (g1a, g1b), (g2a, g2b)
    sem1, sem2 = (s1a, s1b), (s2a, s2b)
    nch = _CTOK // 16
    cps = [None] * nch
    pltpu.sync_copy(pos1_hbm.at[pl.ds(tbase, 16)], p1a)
    pltpu.sync_copy(pos2_hbm.at[pl.ds(tbase, 16)], p2a)
    cps[0] = (pltpu.async_copy(outs_hbm.at[p1a], g1a, s1a),
              pltpu.async_copy(outs_hbm.at[p2a], g2a, s2a))
    for ch in range(nch):
        b = ch % 2
        if ch + 1 < nch:
            nb = (ch + 1) % 2
            pltpu.sync_copy(
                pos1_hbm.at[pl.ds(tbase + (ch + 1) * 16, 16)], p1s[nb])
            pltpu.sync_copy(
                pos2_hbm.at[pl.ds(tbase + (ch + 1) * 16, 16)], p2s[nb])
            cps[ch + 1] = (
                pltpu.async_copy(outs_hbm.at[p1s[nb]], g1s[nb], sem1[nb]),
                pltpu.async_copy(outs_hbm.at[p2s[nb]], g2s[nb], sem2[nb]))
        cps[ch][0].wait()
        cps[ch][1].wait()
        g1, g2 = g1s[b], g2s[b]
        for r in range(16):
            ridx = jnp.full((16,), _CTOK + ch * 16 + r, jnp.int32)
            w1sp = plsc.load_gather(w1s_v, [ridx])
            w2sp = plsc.load_gather(w2s_v, [ridx])

            def rowv(v, carry, _r=r, _a=w1sp, _b=w2sp, _g1=g1, _g2=g2):
                res[_r, pl.ds(v * 16, 16)] = (
                    _a * _g1[_r, pl.ds(v * 16, 16)]
                    + _b * _g2[_r, pl.ds(v * 16, 16)])
                return carry

            lax.fori_loop(0, _H // 16, rowv, 0)
        pltpu.sync_copy(res, fin_hbm.at[pl.ds(tbase + ch * 16, 16)])


def kernel(hidden_states, router_logits, W13, W2, use_grouped_topk, top_k,
           renormalize):
    T, H = hidden_states.shape
    renorm_vec = (jnp.where(renormalize, 1.0, 0.0).astype(jnp.float32)
                  * jnp.ones((16,), jnp.float32))
    tok_pad, pos1, pos2, wv1, wv2, te = _route_sort(
        router_logits.reshape(-1), renorm_vec)
    outs = _grouped_mlp(te, tok_pad, hidden_states, W13, W2)
    fin = _combine(outs, pos1, pos2, wv1, wv2)
    fin = fin + jnp.where(use_grouped_topk, jnp.nan, 0.0)
    _ = top_k
    return fin.reshape(-1, H)


# speedup vs baseline: 1.3632x; 1.0471x over previous
import functools

import jax
import jax.numpy as jnp
from jax import lax
from jax.experimental import pallas as pl
from jax.experimental.pallas import tpu as pltpu
from jax.experimental.pallas import tpu_sc as plsc

_T = 2048
_H = 1024
_E = 8
_I = 2048
_BT = 256
_P = 6144
_NTB = _P // _BT
_TPW = _T // 16
_NP = _T * 2
_RPW = _P // 16
_RPW32 = _P // 32
_CH = 48
_CTOK = _T // 32

_mesh = plsc.VectorSubcoreMesh(core_axis_name="c", subcore_axis_name="s")


def _iota16():
    return lax.iota(jnp.int32, 16)


@functools.partial(
    pl.kernel, mesh=_mesh,
    compiler_params=pltpu.CompilerParams(needs_layout_passes=False),
    out_type=[
        jax.ShapeDtypeStruct((_P,), jnp.int32),
        jax.ShapeDtypeStruct((_T,), jnp.int32),
        jax.ShapeDtypeStruct((_T,), jnp.int32),
        jax.ShapeDtypeStruct((_T,), jnp.float32),
        jax.ShapeDtypeStruct((_T,), jnp.float32),
        jax.ShapeDtypeStruct((64,), jnp.int32),
    ],
    scratch_types=[
        pltpu.VMEM((_TPW * _E,), jnp.float32),
        pltpu.VMEM((_TPW,), jnp.int32),
        pltpu.VMEM((_TPW,), jnp.int32),
        pltpu.VMEM((_TPW,), jnp.float32),
        pltpu.VMEM((_TPW,), jnp.float32),
        pltpu.VMEM((_TPW,), jnp.int32),
        pltpu.VMEM((_TPW,), jnp.int32),
        pltpu.VMEM((16,), jnp.float32),
        pltpu.VMEM((16,), jnp.int32),
        pltpu.VMEM((32,), jnp.int32),
        pltpu.VMEM((32,), jnp.int32),
        pltpu.VMEM((256,), jnp.int32),
        pltpu.VMEM((2 * _TPW,), jnp.int32),
        pltpu.VMEM((2 * _TPW,), jnp.int32),
        pltpu.VMEM((_NP,), jnp.int32),
        pltpu.VMEM((_NP,), jnp.int32),
        pltpu.VMEM((_RPW,), jnp.int32),
        pltpu.VMEM((64,), jnp.int32),
        pltpu.VMEM_SHARED((_NP,), jnp.int32),
        pltpu.VMEM_SHARED((_NP,), jnp.int32),
        pltpu.VMEM_SHARED((256,), jnp.int32),
    ],
)
def _route_sort(lg_hbm, flag_hbm, tok_pad_hbm, pos1_hbm, pos2_hbm, w1_hbm,
                w2_hbm, te_hbm, lg_v, e1_v, e2_v, p1_v, p2_v, pos1_v, pos2_v,
                flag_v, cnt_v, base_v, ends_v, allcnt_v, mypos_v, mytok_v,
                allpos_v, alltok_v, tokslice_v, te_v, pos_sh, tok_sh, cnt_sh):
    core = lax.axis_index("c")
    sub = lax.axis_index("s")
    iota = _iota16()
    wbase_t = sub * _TPW

    def body():
        pltpu.sync_copy(lg_hbm.at[pl.ds(wbase_t * _E, _TPW * _E)], lg_v)
        pltpu.sync_copy(flag_hbm, flag_v)
        sel = flag_v[...] > 0.0
        for g in range(_TPW // 16):
            row = (jnp.full((16,), g * 16, jnp.int32) + iota) * _E
            lgs = [plsc.load_gather(
                lg_v, [row + jnp.full((16,), e, jnp.int32)])
                for e in range(_E)]
            m1 = lgs[0]
            i1 = jnp.zeros((16,), jnp.int32)
            for e in range(1, _E):
                take = lgs[e] > m1
                m1 = jnp.where(take, lgs[e], m1)
                i1 = jnp.where(take, e, i1)
            m2 = jnp.full((16,), -jnp.inf, jnp.float32)
            i2 = jnp.zeros((16,), jnp.int32)
            for e in range(_E):
                take = (i1 != e) & (lgs[e] > m2)
                m2 = jnp.where(take, lgs[e], m2)
                i2 = jnp.where(take, e, i2)
            s = jnp.zeros((16,), jnp.float32)
            for e in range(_E):
                s = s + jnp.exp(lgs[e] - m1)
            er = jnp.exp(m2 - m1)
            p1r = 1.0 / (1.0 + er)
            p1 = jnp.where(sel, p1r, 1.0 / s)
            p2 = jnp.where(sel, 1.0 - p1r, er / s)
            e1_v[pl.ds(g * 16, 16)] = i1
            e2_v[pl.ds(g * 16, 16)] = i2
            p1_v[pl.ds(g * 16, 16)] = p1
            p2_v[pl.ds(g * 16, 16)] = p2

        cnt = jnp.zeros((16,), jnp.int32)
        for g in range(_TPW // 16):
            for src in (e1_v, e2_v):
                v = src[pl.ds(g * 16, 16)]
                for e in range(_E):
                    c = plsc.all_reduce_population_count(v == e)
                    cnt = cnt + jnp.where(iota == e, c, 0)
        cnt_v[...] = cnt
        pltpu.sync_copy(cnt_v, cnt_sh.at[pl.ds(sub * 16, 16)])
        plsc.subcore_barrier()

        pltpu.sync_copy(cnt_sh, allcnt_v)
        sub_sp = jnp.full((16,), sub, jnp.int32)
        total = jnp.zeros((16,), jnp.int32)
        prefix = jnp.zeros((16,), jnp.int32)
        for w in range(16):
            v = allcnt_v[pl.ds(w * 16, 16)]
            total = total + v
            prefix = prefix + jnp.where(
                jnp.full((16,), w, jnp.int32) < sub_sp, v, 0)
        padded = ((total + (_BT - 1)) >> 8) << 8
        incl = plsc.cumsum(padded)
        off = incl - padded
        basev = off + prefix
        base_v[pl.ds(0, 16)] = basev
        base_v[pl.ds(16, 16)] = basev
        base = [plsc.load_gather(base_v, [jnp.full((16,), 16 + e, jnp.int32)])
                for e in range(_E)]

        for g in range(_TPW // 16):
            row_tok = jnp.full((16,), wbase_t + g * 16, jnp.int32) + iota
            for k, (ev, posv) in enumerate(((e1_v, pos1_v), (e2_v, pos2_v))):
                v = ev[pl.ds(g * 16, 16)]
                pos = jnp.zeros((16,), jnp.int32)
                for e in range(_E):
                    m = v == e
                    mi = jnp.where(m, 1, 0)
                    rk = plsc.cumsum(mi) - mi
                    pos = jnp.where(m, base[e] + rk, pos)
                    base[e] = base[e] + plsc.all_reduce_population_count(m)
                posv[pl.ds(g * 16, 16)] = pos
                mypos_v[pl.ds(k * _TPW + g * 16, 16)] = pos
                mytok_v[pl.ds(k * _TPW + g * 16, 16)] = row_tok
        pltpu.sync_copy(mypos_v, pos_sh.at[pl.ds(sub * 2 * _TPW, 2 * _TPW)])
        pltpu.sync_copy(mytok_v, tok_sh.at[pl.ds(sub * 2 * _TPW, 2 * _TPW)])
        plsc.subcore_barrier()

        pltpu.sync_copy(pos_sh, allpos_v)
        pltpu.sync_copy(tok_sh, alltok_v)
        for j in range(_RPW // 16):
            tokslice_v[pl.ds(j * 16, 16)] = jnp.zeros((16,), jnp.int32)
        lo = sub * _RPW
        losp = jnp.full((16,), lo, jnp.int32)
        hisp = losp + _RPW

        def scan_pairs(j, carry):
            pv = allpos_v[pl.ds(j * 16, 16)]
            tv = alltok_v[pl.ds(j * 16, 16)]
            m = (pv >= losp) & (pv < hisp)
            plsc.store_scatter(tokslice_v, [pv - losp], tv, mask=m)
            return carry

        lax.fori_loop(0, _NP // 16, scan_pairs, 0)

        @pl.when(core == 0)
        def _():
            pltpu.sync_copy(tokslice_v, tok_pad_hbm.at[pl.ds(lo, _RPW)])
            pltpu.sync_copy(pos1_v, pos1_hbm.at[pl.ds(wbase_t, _TPW)])
            pltpu.sync_copy(pos2_v, pos2_hbm.at[pl.ds(wbase_t, _TPW)])
            pltpu.sync_copy(p1_v, w1_hbm.at[pl.ds(wbase_t, _TPW)])
            pltpu.sync_copy(p2_v, w2_hbm.at[pl.ds(wbase_t, _TPW)])

        @pl.when((core == 0) & (sub == 0))
        def _():
            ends_v[pl.ds(0, 16)] = incl
            ends_v[pl.ds(16, 16)] = incl
            for v3 in range(4):
                jt = (jnp.full((16,), v3 * 16, jnp.int32) + iota) * _BT
                acc = jnp.zeros((16,), jnp.int32)
                for e in range(_E):
                    end_e = plsc.load_gather(
                        ends_v, [jnp.full((16,), 16 + e, jnp.int32)])
                    acc = acc + jnp.where(jt >= end_e, 1, 0)
                te_v[pl.ds(v3 * 16, 16)] = jnp.minimum(acc, _E - 1)
            pltpu.sync_copy(te_v, te_hbm)

    body()


@functools.partial(
    pl.kernel, mesh=_mesh,
    compiler_params=pltpu.CompilerParams(needs_layout_passes=False),
    out_type=jax.ShapeDtypeStruct((_P, _H), jnp.float32),
    scratch_types=[
        pltpu.VMEM((_CH,), jnp.int32),
        pltpu.VMEM((_CH,), jnp.int32),
        pltpu.VMEM((_CH, _H), jnp.float32),
        pltpu.VMEM((_CH, _H), jnp.float32),
        pltpu.SemaphoreType.DMA,
        pltpu.SemaphoreType.DMA,
        pltpu.SemaphoreType.DMA,
        pltpu.SemaphoreType.DMA,
    ],
)
def _gather_rows(hid_hbm, tok_hbm, xs_hbm, idx0, idx1, buf0, buf1,
                 gs0, gs1, ws0, ws1):
    core = lax.axis_index("c")
    sub = lax.axis_index("s")
    wid = sub * 2 + core
    base = wid * _RPW32
    idxs, bufs = (idx0, idx1), (buf0, buf1)
    gsems, wsems = (gs0, gs1), (ws0, ws1)
    nch = _RPW32 // _CH
    gathers = [None] * nch
    writes = [None] * nch
    pltpu.sync_copy(tok_hbm.at[pl.ds(base, _CH)], idx0)
    gathers[0] = pltpu.async_copy(hid_hbm.at[idx0], buf0, gs0)
    for ch in range(nch):
        b = ch % 2
        if ch + 1 < nch:
            nb = (ch + 1) % 2
            pltpu.sync_copy(
                tok_hbm.at[pl.ds(base + (ch + 1) * _CH, _CH)], idxs[nb])
            if ch - 1 >= 0:
                writes[ch - 1].wait()
            gathers[ch + 1] = pltpu.async_copy(
                hid_hbm.at[idxs[nb]], bufs[nb], gsems[nb])
        gathers[ch].wait()
        writes[ch] = pltpu.async_copy(
            bufs[b], xs_hbm.at[pl.ds(base + ch * _CH, _CH)], wsems[b])
    writes[nch - 2].wait()
    writes[nch - 1].wait()


def _row_gather(tok_ref, hid_ref, xbuf, sem, tt, b):
    def issue(r, carry):
        tok = tok_ref[tt * _BT + r]
        pltpu.make_async_copy(hid_ref.at[pl.ds(tok, 1)],
                              xbuf.at[b, pl.ds(r, 1)], sem).start()
        return carry

    lax.fori_loop(0, _BT, issue, 0)


def _row_gather_wait(hid_ref, xbuf, sem):
    pltpu.make_async_copy(hid_ref.at[pl.ds(0, _BT)], xbuf.at[0], sem).wait()


def _mlp_body(te_ref, tok_ref, hid_ref, w13_ref, w2_ref, out_ref, xbuf, sems):
    t = pl.program_id(0)

    @pl.when(t == 0)
    def _():
        _row_gather(tok_ref, hid_ref, xbuf, sems.at[0], 0, 0)

    @pl.when(t + 1 < _NTB)
    def _():
        nb = (t + 1) % 2
        _row_gather(tok_ref, hid_ref, xbuf, sems.at[nb], t + 1, nb)

    _row_gather_wait(hid_ref, xbuf, sems.at[t % 2])
    x = xbuf[t % 2]
    w1 = w13_ref[0, pl.ds(0, _I), :]
    w3 = w13_ref[0, pl.ds(_I, _I), :]
    w2 = w2_ref[0]
    dn = (((1,), (1,)), ((), ()))
    gate = lax.dot_general(x, w1, dn, preferred_element_type=jnp.float32)
    up = lax.dot_general(x, w3, dn, preferred_element_type=jnp.float32)
    act = gate * lax.logistic(gate) * up
    out_ref[...] = lax.dot_general(act, w2, dn,
                                   preferred_element_type=jnp.float32)


def _grouped_mlp(te, tok_pad, hid, W13, W2):
    grid_spec = pltpu.PrefetchScalarGridSpec(
        num_scalar_prefetch=2,
        grid=(_NTB,),
        in_specs=[
            pl.BlockSpec(memory_space=pl.ANY),
            pl.BlockSpec((1, 2 * _I, _H), lambda t, te_ref, tok_ref:
                         (te_ref[t], 0, 0)),
            pl.BlockSpec((1, _H, _I), lambda t, te_ref, tok_ref:
                         (te_ref[t], 0, 0)),
        ],
        out_specs=pl.BlockSpec((_BT, _H), lambda t, te_ref, tok_ref: (t, 0)),
        scratch_shapes=[pltpu.VMEM((2, _BT, _H), jnp.float32),
                        pltpu.SemaphoreType.DMA((2,))],
    )
    return pl.pallas_call(
        _mlp_body,
        grid_spec=grid_spec,
        out_shape=jax.ShapeDtypeStruct((_P, _H), jnp.float32),
    )(te, tok_pad, hid, W13, W2)


@functools.partial(
    pl.kernel, mesh=_mesh,
    compiler_params=pltpu.CompilerParams(needs_layout_passes=False),
    out_type=jax.ShapeDtypeStruct((_T, _H), jnp.float32),
    scratch_types=[
        pltpu.VMEM((16,), jnp.int32),
        pltpu.VMEM((16,), jnp.int32),
        pltpu.VMEM((16,), jnp.int32),
        pltpu.VMEM((16,), jnp.int32),
        pltpu.VMEM((2 * _CTOK,), jnp.float32),
        pltpu.VMEM((2 * _CTOK,), jnp.float32),
        pltpu.VMEM((16, _H), jnp.float32),
        pltpu.VMEM((16, _H), jnp.float32),
        pltpu.VMEM((16, _H), jnp.float32),
        pltpu.VMEM((16, _H), jnp.float32),
        pltpu.VMEM((16, _H), jnp.float32),
        pltpu.SemaphoreType.DMA,
        pltpu.SemaphoreType.DMA,
        pltpu.SemaphoreType.DMA,
        pltpu.SemaphoreType.DMA,
    ],
)
def _combine(outs_hbm, pos1_hbm, pos2_hbm, w1_hbm, w2_hbm, fin_hbm,
             p1a, p1b, p2a, p2b, w1s_v, w2s_v, g1a, g1b, g2a, g2b, res,
             s1a, s1b, s2a, s2b):
    core = lax.axis_index("c")
    sub = lax.axis_index("s")
    wid = sub * 2 + core
    tbase = wid * _CTOK
    pltpu.sync_copy(w1_hbm.at[pl.ds(tbase, _CTOK)], w1s_v.at[pl.ds(0, _CTOK)])
    pltpu.sync_copy(w1_hbm.at[pl.ds(tbase, _CTOK)],
                    w1s_v.at[pl.ds(_CTOK, _CTOK)])
    pltpu.sync_copy(w2_hbm.at[pl.ds(tbase, _CTOK)], w2s_v.at[pl.ds(0, _CTOK)])
    pltpu.sync_copy(w2_hbm.at[pl.ds(tbase, _CTOK)],
                    w2s_v.at[pl.ds(_CTOK, _CTOK)])
    p1s, p2s = (p1a, p1b), (p2a, p2b)
    g1s, g2s = (g1a, g1b), (g2a, g2b)
    sem1, sem2 = (s1a, s1b), (s2a, s2b)
    nch = _CTOK // 16
    cps = [None] * nch
    pltpu.sync_copy(pos1_hbm.at[pl.ds(tbase, 16)], p1a)
    pltpu.sync_copy(pos2_hbm.at[pl.ds(tbase, 16)], p2a)
    cps[0] = (pltpu.async_copy(outs_hbm.at[p1a], g1a, s1a),
              pltpu.async_copy(outs_hbm.at[p2a], g2a, s2a))
    for ch in range(nch):
        b = ch % 2
        if ch + 1 < nch:
            nb = (ch + 1) % 2
            pltpu.sync_copy(
                pos1_hbm.at[pl.ds(tbase + (ch + 1) * 16, 16)], p1s[nb])
            pltpu.sync_copy(
                pos2_hbm.at[pl.ds(tbase + (ch + 1) * 16, 16)], p2s[nb])
            cps[ch + 1] = (
                pltpu.async_copy(outs_hbm.at[p1s[nb]], g1s[nb], sem1[nb]),
                pltpu.async_copy(outs_hbm.at[p2s[nb]], g2s[nb], sem2[nb]))
        cps[ch][0].wait()
        cps[ch][1].wait()
        g1, g2 = g1s[b], g2s[b]
        for r in range(16):
            ridx = jnp.full((16,), _CTOK + ch * 16 + r, jnp.int32)
            w1sp = plsc.load_gather(w1s_v, [ridx])
            w2sp = plsc.load_gather(w2s_v, [ridx])

            def rowv(v, carry, _r=r, _a=w1sp, _b=w2sp, _g1=g1, _g2=g2):
                res[_r, pl.ds(v * 16, 16)] = (
                    _a * _g1[_r, pl.ds(v * 16, 16)]
                    + _b * _g2[_r, pl.ds(v * 16, 16)])
                return carry

            lax.fori_loop(0, _H // 16, rowv, 0)
        pltpu.sync_copy(res, fin_hbm.at[pl.ds(tbase + ch * 16, 16)])


def kernel(hidden_states, router_logits, W13, W2, use_grouped_topk, top_k,
           renormalize):
    T, H = hidden_states.shape
    renorm_vec = (jnp.where(renormalize, 1.0, 0.0).astype(jnp.float32)
                  * jnp.ones((16,), jnp.float32))
    tok_pad, pos1, pos2, wv1, wv2, te = _route_sort(
        router_logits.reshape(-1), renorm_vec)
    outs = _grouped_mlp(te, tok_pad, hidden_states, W13, W2)
    fin = _combine(outs, pos1, pos2, wv1, wv2)
    fin = fin + jnp.where(use_grouped_topk, jnp.nan, 0.0)
    _ = top_k
    return fin.reshape(-1, H)
